# baseline (speedup 1.0000x reference)
import functools

import jax
import jax.numpy as jnp
from jax import lax
from jax.experimental import pallas as pl
from jax.experimental.pallas import tpu as pltpu
from jax.experimental.pallas import tpu_sc as plsc

VOCAB = 100000
EMBED_DIM = 128
BATCH = 4096
SEQ_LEN = 200

NC = 2
NS = 16
NW = NC * NS

TOTAL = BATCH * SEQ_LEN
PER_W = TOTAL // NW
CHUNK = 128
NSTEPS = PER_W // CHUNK


NSLOT = 4
PAIR = 2


def _body(x_hbm, table_hbm, out_hbm, idx_v, rows_v, sg, sw):
    wid = lax.axis_index("s") * NC + lax.axis_index("c")
    base = wid * PER_W
    pltpu.sync_copy(x_hbm.at[wid], idx_v)

    def gather(j, s):
        pltpu.async_copy(
            table_hbm.at[idx_v.at[j]],
            rows_v.at[pl.ds(s * CHUNK, CHUNK)], sg[s])

    def wait_gather(j, s):
        pltpu.make_async_copy(
            table_hbm.at[idx_v.at[j]],
            rows_v.at[pl.ds(s * CHUNK, CHUNK)], sg[s]).wait()

    def write_pair(p, h):
        return pltpu.async_copy(
            rows_v.at[pl.ds(h * PAIR * CHUNK, PAIR * CHUNK)],
            out_hbm.at[pl.ds(base + p * PAIR * CHUNK, PAIR * CHUNK)], sw[h])

    def do_pair(p, h):
        wait_gather(PAIR * p, PAIR * h)
        wait_gather(PAIR * p + 1, PAIR * h + 1)
        write_pair(p, h).wait()

    NPAIR = NSTEPS // PAIR
    for s in range(NSLOT):
        gather(s, s)

    def step(g):
        for h in range(2):
            p = g + h
            do_pair(p, h)
            gather(PAIR * p + NSLOT, PAIR * h)
            gather(PAIR * p + NSLOT + 1, PAIR * h + 1)

    pl.loop(0, NPAIR - 2, step=2)(step)

    do_pair(NPAIR - 2, 0)
    do_pair(NPAIR - 1, 1)


@jax.jit
def kernel(x, table):
    x3 = x.reshape(NW, NSTEPS, CHUNK).astype(jnp.int32)
    run = functools.partial(
        pl.kernel,
        out_type=jax.ShapeDtypeStruct((TOTAL, EMBED_DIM), jnp.float32),
        mesh=plsc.VectorSubcoreMesh(core_axis_name="c", subcore_axis_name="s"),
        scratch_types=[
            pltpu.VMEM((NSTEPS, CHUNK), jnp.int32),
            pltpu.VMEM((NSLOT * CHUNK, EMBED_DIM), jnp.float32),
            [pltpu.SemaphoreType.DMA] * NSLOT,
            [pltpu.SemaphoreType.DMA] * 2,
        ],
    )(_body)
    out = run(x3, table)
    return out.reshape(BATCH, SEQ_LEN, EMBED_DIM)


# speedup vs baseline: 1.0022x; 1.0022x over previous
import functools

import jax
import jax.numpy as jnp
from jax import lax
from jax.experimental import pallas as pl
from jax.experimental.pallas import tpu as pltpu
from jax.experimental.pallas import tpu_sc as plsc

VOCAB = 100000
EMBED_DIM = 128
BATCH = 4096
SEQ_LEN = 200

NC = 2
NS = 16
NW = NC * NS

TOTAL = BATCH * SEQ_LEN
PER_W = TOTAL // NW
CHUNK = 128
NSTEPS = PER_W // CHUNK
GROUP = 10
NGROUP = NSTEPS // GROUP
SLOT_ROWS = 2 * CHUNK


def _body(x_hbm, table_hbm, out_hbm, idx_v, rows_v, spm, sgat, sxb, sdma, sdw):
    sid = lax.axis_index("s")
    wid = sid * NC + lax.axis_index("c")
    base = wid * PER_W
    pltpu.sync_copy(x_hbm.at[wid], idx_v)
    my_spm = spm.at[sid]

    def gather(c, t):
        pltpu.async_copy(
            table_hbm.at[idx_v.at[c]],
            rows_v.at[pl.ds(t * CHUNK, CHUNK)], sgat[t])

    def wait_gather(c, t):
        pltpu.make_async_copy(
            table_hbm.at[idx_v.at[c]],
            rows_v.at[pl.ds(t * CHUNK, CHUNK)], sgat[t]).wait()

    def direct_write(c, t):
        pltpu.async_copy(
            rows_v.at[pl.ds(t * CHUNK, CHUNK)],
            out_hbm.at[pl.ds(base + c * CHUNK, CHUNK)], sdw[t]).wait()

    def xbar(t, u, h):
        pltpu.async_copy(
            rows_v.at[pl.ds(t * CHUNK, CHUNK)],
            my_spm.at[pl.ds(u * SLOT_ROWS + h * CHUNK, CHUNK)], sxb[u]).wait()

    def flush(c0, u):
        pltpu.async_copy(
            my_spm.at[pl.ds(u * SLOT_ROWS, SLOT_ROWS)],
            out_hbm.at[pl.ds(base + c0 * CHUNK, SLOT_ROWS)], sdma[u])

    def wait_flush(c0, u):
        pltpu.make_async_copy(
            my_spm.at[pl.ds(u * SLOT_ROWS, SLOT_ROWS)],
            out_hbm.at[pl.ds(base + c0 * CHUNK, SLOT_ROWS)], sdma[u]).wait()

    def group_body(g0, flush_prev, drain):
        for k in range(GROUP):
            c = g0 + k
            t = k % 2
            wait_gather(c, t)
            if k < 2:
                direct_write(c, t)
            else:
                u = ((k - 2) // 2) % 2
                h = k % 2
                if h == 0 and flush_prev[u] is not None:
                    wait_flush(flush_prev[u], u)
                xbar(t, u, h)
                if h == 1:
                    flush(c - 1, u)
                    flush_prev[u] = c - 1
            if not (drain and k >= GROUP - 2):
                gather(c + 2, t)
        return flush_prev

    gather(0, 0)
    gather(1, 1)
    group_body(0, [None, None], drain=False)

    def steady(g0):
        group_body(g0, [g0 - GROUP + 6, g0 - GROUP + 8], drain=False)

    pl.loop(GROUP, (NGROUP - 1) * GROUP, step=GROUP)(steady)

    g0 = (NGROUP - 1) * GROUP
    fp = group_body(g0, [g0 - GROUP + 6, g0 - GROUP + 8], drain=True)
    wait_flush(fp[0], 0)
    wait_flush(fp[1], 1)


@jax.jit
def kernel(x, table):
    x3 = x.reshape(NW, NSTEPS, CHUNK).astype(jnp.int32)
    run = functools.partial(
        pl.kernel,
        out_type=jax.ShapeDtypeStruct((TOTAL, EMBED_DIM), jnp.float32),
        mesh=plsc.VectorSubcoreMesh(core_axis_name="c", subcore_axis_name="s"),
        scratch_types=[
            pltpu.VMEM((NSTEPS, CHUNK), jnp.int32),
            pltpu.VMEM((2 * CHUNK, EMBED_DIM), jnp.float32),
            pltpu.VMEM_SHARED((NS, 2 * SLOT_ROWS, EMBED_DIM), jnp.float32),
            [pltpu.SemaphoreType.DMA] * 2,
            [pltpu.SemaphoreType.DMA] * 2,
            [pltpu.SemaphoreType.DMA] * 2,
            [pltpu.SemaphoreType.DMA] * 2,
        ],
    )(_body)
    out = run(x3, table)
    return out.reshape(BATCH, SEQ_LEN, EMBED_DIM)
